# baseline (device time: 15351 ns/iter reference)
import jax
import jax.numpy as jnp
from jax import lax
from jax.experimental import pallas as pl
from jax.experimental.pallas import tpu as pltpu

N_DEV = 4
B, SQ, D = 2, 128, 512
H_LOC, DH = 8, 64
SCALE = 0.125
CH = (B * SQ) // N_DEV


def kernel(x, Wq, Wo, Wk, Wv):
    def body(x_hbm, wq_hbm, wo_hbm, wk_hbm, wv_hbm, out_hbm,
             xv, wqv, wov, wkv, wvv, attnv, sendbuf, rs_buf, ag_buf, outv,
             load_sems, store_sem, rs_send, rs_recv, ag_send, ag_recv):
        my = lax.axis_index("i")

        cp_x = pltpu.make_async_copy(x_hbm, xv, load_sems.at[0])
        cp_q = pltpu.make_async_copy(wq_hbm, wqv, load_sems.at[1])
        cp_k = pltpu.make_async_copy(wk_hbm, wkv, load_sems.at[2])
        cp_v = pltpu.make_async_copy(wv_hbm, wvv, load_sems.at[3])
        cp_o = pltpu.make_async_copy(wo_hbm, wov, load_sems.at[4])
        for cp in (cp_x, cp_q, cp_k, cp_v, cp_o):
            cp.start()

        barrier = pltpu.get_barrier_semaphore()
        for d in range(1, N_DEV):
            dest = lax.rem(my + d, N_DEV)
            pl.semaphore_signal(
                barrier, inc=1,
                device_id=(dest,), device_id_type=pl.DeviceIdType.MESH,
            )
        pl.semaphore_wait(barrier, N_DEV - 1)

        for cp in (cp_x, cp_q, cp_k, cp_v):
            cp.wait()
        xb = xv[...].reshape(B * SQ, D).astype(jnp.bfloat16)
        q = (jnp.dot(xb, wqv[...].astype(jnp.bfloat16),
                     preferred_element_type=jnp.float32)
             * SCALE).astype(jnp.bfloat16)
        k = jnp.dot(xb, wkv[...].astype(jnp.bfloat16),
                    preferred_element_type=jnp.float32).astype(jnp.bfloat16)
        v = jnp.dot(xb, wvv[...].astype(jnp.bfloat16),
                    preferred_element_type=jnp.float32).astype(jnp.bfloat16)

        nt = (((1,), (1,)), ((), ()))
        for b in range(B):
            rows = slice(b * SQ, (b + 1) * SQ)
            for h in range(H_LOC):
                cols = slice(h * DH, (h + 1) * DH)
                s = lax.dot_general(q[rows, cols], k[rows, cols], nt,
                                    preferred_element_type=jnp.float32)
                p = jnp.exp(s)
                l = jnp.sum(p, axis=-1, keepdims=True)
                o = jnp.dot(p.astype(jnp.bfloat16), v[rows, cols],
                            preferred_element_type=jnp.float32) * (1.0 / l)
                attnv[rows, cols] = o.astype(jnp.bfloat16)

        cp_o.wait()
        partial = jnp.dot(attnv[...], wov[...].astype(jnp.bfloat16),
                          preferred_element_type=jnp.float32)
        pb = partial.astype(jnp.bfloat16)
        for s in range(N_DEV):
            sendbuf[s] = pb[s * CH:(s + 1) * CH, :]

        rs_buf[my] = sendbuf[my]
        rs_rdmas = []
        for d in range(1, N_DEV):
            dest = lax.rem(my + d, N_DEV)
            r = pltpu.make_async_remote_copy(
                src_ref=sendbuf.at[dest],
                dst_ref=rs_buf.at[my],
                send_sem=rs_send.at[dest],
                recv_sem=rs_recv.at[my],
                device_id=(dest,),
                device_id_type=pl.DeviceIdType.MESH,
            )
            r.start()
            rs_rdmas.append(r)
        for d in range(1, N_DEV):
            src = lax.rem(my + N_DEV - d, N_DEV)
            pltpu.make_async_remote_copy(
                src_ref=sendbuf.at[src],
                dst_ref=rs_buf.at[src],
                send_sem=rs_send.at[src],
                recv_sem=rs_recv.at[src],
                device_id=(src,),
                device_id_type=pl.DeviceIdType.MESH,
            ).wait_recv()
        red = ((rs_buf[0].astype(jnp.float32) + rs_buf[1].astype(jnp.float32))
               + (rs_buf[2].astype(jnp.float32)
                  + rs_buf[3].astype(jnp.float32)))

        ag_buf[my] = red.astype(jnp.bfloat16)
        ag_rdmas = []
        for d in range(1, N_DEV):
            dest = lax.rem(my + d, N_DEV)
            r = pltpu.make_async_remote_copy(
                src_ref=ag_buf.at[my],
                dst_ref=ag_buf.at[my],
                send_sem=ag_send.at[dest],
                recv_sem=ag_recv.at[my],
                device_id=(dest,),
                device_id_type=pl.DeviceIdType.MESH,
            )
            r.start()
            ag_rdmas.append(r)
        for d in range(1, N_DEV):
            src = lax.rem(my + N_DEV - d, N_DEV)
            pltpu.make_async_remote_copy(
                src_ref=ag_buf.at[src],
                dst_ref=ag_buf.at[src],
                send_sem=ag_send.at[src],
                recv_sem=ag_recv.at[src],
                device_id=(src,),
                device_id_type=pl.DeviceIdType.MESH,
            ).wait_recv()

        for s in range(N_DEV):
            bidx, r0 = divmod(s * CH, SQ)
            outv[bidx, r0:r0 + CH, :] = ag_buf[s].astype(jnp.float32)

        for r in rs_rdmas:
            r.wait_send()
        for r in ag_rdmas:
            r.wait_send()

        st = pltpu.make_async_copy(outv, out_hbm, store_sem)
        st.start()
        st.wait()

    f32, bf16 = jnp.float32, jnp.bfloat16
    return pl.pallas_call(
        body,
        out_shape=jax.ShapeDtypeStruct((B, SQ, D), f32),
        in_specs=[pl.BlockSpec(memory_space=pltpu.MemorySpace.HBM)] * 5,
        out_specs=pl.BlockSpec(memory_space=pltpu.MemorySpace.HBM),
        scratch_shapes=[
            pltpu.VMEM((B, SQ, D), f32),
            pltpu.VMEM((D, D), f32),
            pltpu.VMEM((D, D), f32),
            pltpu.VMEM((D, D), f32),
            pltpu.VMEM((D, D), f32),
            pltpu.VMEM((B * SQ, H_LOC * DH), bf16),
            pltpu.VMEM((N_DEV, CH, D), bf16),
            pltpu.VMEM((N_DEV, CH, D), bf16),
            pltpu.VMEM((N_DEV, CH, D), bf16),
            pltpu.VMEM((B, SQ, D), f32),
            pltpu.SemaphoreType.DMA((5,)),
            pltpu.SemaphoreType.DMA,
            pltpu.SemaphoreType.DMA((N_DEV,)),
            pltpu.SemaphoreType.DMA((N_DEV,)),
            pltpu.SemaphoreType.DMA((N_DEV,)),
            pltpu.SemaphoreType.DMA((N_DEV,)),
        ],
        compiler_params=pltpu.CompilerParams(collective_id=0),
    )(*(pltpu.with_memory_space_constraint(a, pltpu.MemorySpace.HBM)
        for a in (x, Wq, Wo, Wk, Wv)))


# device time: 13227 ns/iter; 1.1606x vs baseline; 1.1606x over previous
import jax
import jax.numpy as jnp
from jax import lax
from jax.experimental import pallas as pl
from jax.experimental.pallas import tpu as pltpu

N_DEV = 4
B, SQ, D = 2, 128, 512
H_LOC, DH = 8, 64
SCALE = 0.125
CH = (B * SQ) // N_DEV


def kernel(x, Wq, Wo, Wk, Wv):
    def body(x_hbm, wq_hbm, wo_hbm, wk_hbm, wv_hbm, out_hbm,
             xv, wqv, wov, wkv, wvv, sendbuf, rs_buf, ag_buf, outv,
             load_sems, store_sem, rs_send, rs_recv, ag_send, ag_recv):
        my = lax.axis_index("i")

        cp_x = pltpu.make_async_copy(x_hbm, xv, load_sems.at[0])
        cp_q = pltpu.make_async_copy(wq_hbm, wqv, load_sems.at[1])
        cp_k = pltpu.make_async_copy(wk_hbm, wkv, load_sems.at[2])
        cp_v = pltpu.make_async_copy(wv_hbm, wvv, load_sems.at[3])
        cp_o = pltpu.make_async_copy(wo_hbm, wov, load_sems.at[4])
        for cp in (cp_x, cp_q, cp_k, cp_v, cp_o):
            cp.start()

        barrier = pltpu.get_barrier_semaphore()
        for d in range(1, N_DEV):
            dest = lax.rem(my + d, N_DEV)
            pl.semaphore_signal(
                barrier, inc=1,
                device_id=(dest,), device_id_type=pl.DeviceIdType.MESH,
            )

        cp_x.wait()
        cp_q.wait()
        xb = xv[...].reshape(B * SQ, D).astype(jnp.bfloat16)
        wqb = (wqv[...] * SCALE).astype(jnp.bfloat16)
        q = jnp.dot(xb, wqb,
                    preferred_element_type=jnp.float32).astype(jnp.bfloat16)
        cp_k.wait()
        k = jnp.dot(xb, wkv[...].astype(jnp.bfloat16),
                    preferred_element_type=jnp.float32).astype(jnp.bfloat16)
        cp_v.wait()
        v = jnp.dot(xb, wvv[...].astype(jnp.bfloat16),
                    preferred_element_type=jnp.float32).astype(jnp.bfloat16)

        def to_heads(t):
            return t.reshape(B, SQ, H_LOC, DH).transpose(0, 2, 1, 3).reshape(
                B * H_LOC, SQ, DH)

        q3, k3, v3 = to_heads(q), to_heads(k), to_heads(v)
        s3 = lax.dot_general(q3, k3, (((2,), (2,)), ((0,), (0,))),
                             preferred_element_type=jnp.float32)
        p3 = jnp.exp(s3)
        l3 = jnp.sum(p3, axis=-1, keepdims=True)
        o3 = lax.dot_general(p3.astype(jnp.bfloat16), v3,
                             (((2,), (1,)), ((0,), (0,))),
                             preferred_element_type=jnp.float32) * (1.0 / l3)
        av = o3.astype(jnp.bfloat16).reshape(
            B, H_LOC, SQ, DH).transpose(0, 2, 1, 3).reshape(B * SQ, H_LOC * DH)

        cp_o.wait()
        wob = wov[...].astype(jnp.bfloat16)
        pl.semaphore_wait(barrier, N_DEV - 1)

        def wo_and_send(m):
            for d in range(1, N_DEV + 1):
                s = (m + d) % N_DEV
                pc = jnp.dot(av[s * CH:(s + 1) * CH, :], wob,
                             preferred_element_type=jnp.float32)
                sendbuf[s] = pc.astype(jnp.bfloat16)
                if s != m:
                    pltpu.make_async_remote_copy(
                        src_ref=sendbuf.at[s],
                        dst_ref=rs_buf.at[m],
                        send_sem=rs_send.at[s],
                        recv_sem=rs_recv.at[m],
                        device_id=(s,),
                        device_id_type=pl.DeviceIdType.MESH,
                    ).start()
                else:
                    rs_buf[m] = pc.astype(jnp.bfloat16)

        for m in range(N_DEV):
            @pl.when(my == m)
            def _(m=m):
                wo_and_send(m)
        for d in range(1, N_DEV):
            src = lax.rem(my + N_DEV - d, N_DEV)
            pltpu.make_async_remote_copy(
                src_ref=sendbuf.at[src],
                dst_ref=rs_buf.at[src],
                send_sem=rs_send.at[src],
                recv_sem=rs_recv.at[src],
                device_id=(src,),
                device_id_type=pl.DeviceIdType.MESH,
            ).wait_recv()
        red = ((rs_buf[0].astype(jnp.float32) + rs_buf[1].astype(jnp.float32))
               + (rs_buf[2].astype(jnp.float32)
                  + rs_buf[3].astype(jnp.float32)))

        ag_buf[my] = red.astype(jnp.bfloat16)
        ag_rdmas = []
        for d in range(1, N_DEV):
            dest = lax.rem(my + d, N_DEV)
            r = pltpu.make_async_remote_copy(
                src_ref=ag_buf.at[my],
                dst_ref=ag_buf.at[my],
                send_sem=ag_send.at[dest],
                recv_sem=ag_recv.at[my],
                device_id=(dest,),
                device_id_type=pl.DeviceIdType.MESH,
            )
            r.start()
            ag_rdmas.append(r)
        for d in range(1, N_DEV):
            src = lax.rem(my + N_DEV - d, N_DEV)
            pltpu.make_async_remote_copy(
                src_ref=ag_buf.at[src],
                dst_ref=ag_buf.at[src],
                send_sem=ag_send.at[src],
                recv_sem=ag_recv.at[src],
                device_id=(src,),
                device_id_type=pl.DeviceIdType.MESH,
            ).wait_recv()

        for s in range(N_DEV):
            bidx, r0 = divmod(s * CH, SQ)
            outv[bidx, r0:r0 + CH, :] = ag_buf[s]

        for s in range(N_DEV):
            @pl.when(my != s)
            def _():
                pltpu.make_async_remote_copy(
                    src_ref=sendbuf.at[s],
                    dst_ref=rs_buf.at[my],
                    send_sem=rs_send.at[s],
                    recv_sem=rs_recv.at[my],
                    device_id=(s,),
                    device_id_type=pl.DeviceIdType.MESH,
                ).wait_send()
        for r in ag_rdmas:
            r.wait_send()

        st = pltpu.make_async_copy(outv, out_hbm, store_sem)
        st.start()
        st.wait()

    f32, bf16 = jnp.float32, jnp.bfloat16
    return pl.pallas_call(
        body,
        out_shape=jax.ShapeDtypeStruct((B, SQ, D), bf16),
        in_specs=[pl.BlockSpec(memory_space=pltpu.MemorySpace.HBM)] * 5,
        out_specs=pl.BlockSpec(memory_space=pltpu.MemorySpace.HBM),
        scratch_shapes=[
            pltpu.VMEM((B, SQ, D), f32),
            pltpu.VMEM((D, D), f32),
            pltpu.VMEM((D, D), f32),
            pltpu.VMEM((D, D), f32),
            pltpu.VMEM((D, D), f32),
            pltpu.VMEM((N_DEV, CH, D), bf16),
            pltpu.VMEM((N_DEV, CH, D), bf16),
            pltpu.VMEM((N_DEV, CH, D), bf16),
            pltpu.VMEM((B, SQ, D), bf16),
            pltpu.SemaphoreType.DMA((5,)),
            pltpu.SemaphoreType.DMA,
            pltpu.SemaphoreType.DMA((N_DEV,)),
            pltpu.SemaphoreType.DMA((N_DEV,)),
            pltpu.SemaphoreType.DMA((N_DEV,)),
            pltpu.SemaphoreType.DMA((N_DEV,)),
        ],
        compiler_params=pltpu.CompilerParams(collective_id=0),
    )(*(pltpu.with_memory_space_constraint(a, pltpu.MemorySpace.HBM)
        for a in (x, Wq, Wo, Wk, Wv)))
